# Initial kernel scaffold; baseline (speedup 1.0000x reference)
import jax
import jax.numpy as jnp
from jax.experimental import pallas as pl


def kernel(indices, weight, bias):
    raise NotImplementedError("write your pallas kernel here")


# speedup vs baseline: 4.1295x; 4.1295x over previous
import functools

import jax
import jax.numpy as jnp
from jax import lax
from jax.experimental import pallas as pl
from jax.experimental.pallas import tpu as pltpu
from jax.experimental.pallas import tpu_sc as plsc


def _round_up(x, m):
    return (x + m - 1) // m * m


def _make_kernel(B, S, D, C):
    try:
        info = plsc.get_sparse_core_info()
        NC, NS, L = info.num_cores, info.num_subcores, info.num_lanes
    except ValueError:
        NC, NS, L = 2, 16, 16
    NW = NC * NS
    assert B % NW == 0
    BW = B // NW
    assert D % L == 0
    UD = D // L
    SP = _round_up(S, 8)

    mesh = plsc.VectorSubcoreMesh(core_axis_name="c", subcore_axis_name="s",
                                  num_cores=NC, num_subcores=NS)

    @functools.partial(
        pl.kernel,
        out_type=jax.ShapeDtypeStruct((B, D), jnp.float32),
        mesh=mesh,
        scratch_types=[
            pltpu.VMEM((BW, S), jnp.int32),
            pltpu.VMEM((BW * SP,), jnp.int32),
            pltpu.VMEM((S, D), jnp.float32),
            pltpu.VMEM((S, D), jnp.float32),
            pltpu.VMEM((BW, D), jnp.float32),
            pltpu.VMEM((D,), jnp.float32),
            pltpu.SemaphoreType.DMA,
            pltpu.SemaphoreType.DMA,
        ],
    )
    def k(idx_hbm, w_hbm, bias_hbm, out_hbm,
          raw_v, ids_v, rows0, rows1, acc_v, bias_v, sem0, sem1):
        wid = lax.axis_index("s") * NC + lax.axis_index("c")
        base = wid * BW
        pltpu.sync_copy(idx_hbm.at[pl.ds(base, BW)], raw_v)
        pltpu.sync_copy(bias_hbm, bias_v)

        lane = lax.iota(jnp.int32, L)

        starts = [v * L for v in range(S // L)]
        if S % L:
            starts.append(S - L)

        def tok_body(j, carry):
            for p0 in starts:
                tok = raw_v[j, pl.ds(p0, L)] + (lane + p0) * C
                ids_v[pl.ds(j * SP + p0, L)] = tok
            return carry

        lax.fori_loop(0, BW, tok_body, 0)

        def fire(j, buf, sem):
            pltpu.async_copy(w_hbm.at[ids_v.at[pl.ds(j * SP, S)]], buf, sem)

        def wait(buf, sem):
            pltpu.make_async_copy(w_hbm.at[ids_v.at[pl.ds(0, S)]], buf, sem).wait()

        def accum(j, buf):
            for u in range(UD):
                acc_v[j, pl.ds(u * L, L)] = bias_v[pl.ds(u * L, L)]

            def body(r, carry):
                for u in range(UD):
                    plsc.addupdate(acc_v.at[j, pl.ds(u * L, L)],
                                   buf[r, pl.ds(u * L, L)])
                return carry

            lax.fori_loop(0, S, body, 0)

        fire(0, rows0, sem0)
        fire(1, rows1, sem1)

        def bag_body(t, carry):
            wait(rows0, sem0)
            accum(2 * t, rows0)

            @pl.when(t < BW // 2 - 1)
            def _():
                fire(2 * t + 2, rows0, sem0)

            wait(rows1, sem1)
            accum(2 * t + 1, rows1)

            @pl.when(t < BW // 2 - 1)
            def _():
                fire(2 * t + 3, rows1, sem1)

            return carry

        lax.fori_loop(0, BW // 2, bag_body, 0)

        pltpu.sync_copy(acc_v, out_hbm.at[pl.ds(base, BW)])

    return k


def kernel(indices, weight, bias):
    B, S = indices.shape
    V, D = weight.shape
    C = V // S
    k = _make_kernel(B, S, D, C)
    return k(indices.astype(jnp.int32), weight, bias)


# speedup vs baseline: 16.3598x; 3.9617x over previous
import functools

import jax
import jax.numpy as jnp
from jax import lax
from jax.experimental import pallas as pl
from jax.experimental.pallas import tpu as pltpu
from jax.experimental.pallas import tpu_sc as plsc

_NBUF = 4


def _round_up(x, m):
    return (x + m - 1) // m * m


def _make_kernel(B, S, D, C):
    try:
        info = plsc.get_sparse_core_info()
        NC, NS, L = info.num_cores, info.num_subcores, info.num_lanes
    except ValueError:
        NC, NS, L = 2, 16, 16
    NW = NC * NS
    assert B % NW == 0
    BW = B // NW
    assert D % L == 0
    UD = D // L
    SP = _round_up(S, 8)
    assert BW % _NBUF == 0

    mesh = plsc.VectorSubcoreMesh(core_axis_name="c", subcore_axis_name="s",
                                  num_cores=NC, num_subcores=NS)

    @functools.partial(
        pl.kernel,
        out_type=jax.ShapeDtypeStruct((B, D), jnp.float32),
        mesh=mesh,
        scratch_types=[
            pltpu.VMEM((BW, S), jnp.int32),
            pltpu.VMEM((BW * SP,), jnp.int32),
            [pltpu.VMEM((S, D), jnp.float32) for _ in range(_NBUF)],
            pltpu.VMEM((BW, D), jnp.float32),
            pltpu.VMEM((D,), jnp.float32),
            [pltpu.SemaphoreType.DMA for _ in range(_NBUF)],
        ],
    )
    def k(idx_hbm, w_hbm, bias_hbm, out_hbm,
          raw_v, ids_v, rows, acc_v, bias_v, sems):
        wid = lax.axis_index("s") * NC + lax.axis_index("c")
        base = wid * BW
        pltpu.sync_copy(idx_hbm.at[pl.ds(base, BW)], raw_v)
        pltpu.sync_copy(bias_hbm, bias_v)

        lane = lax.iota(jnp.int32, L)

        starts = [v * L for v in range(S // L)]
        if S % L:
            starts.append(S - L)

        def tok_body(j, carry):
            for p0 in starts:
                tok = raw_v[j, pl.ds(p0, L)] + (lane + p0) * C
                ids_v[pl.ds(j * SP + p0, L)] = tok
            return carry

        lax.fori_loop(0, BW, tok_body, 0)

        def fire(j, buf, sem):
            pltpu.async_copy(w_hbm.at[ids_v.at[pl.ds(j * SP, S)]], buf, sem)

        def wait(buf, sem):
            pltpu.make_async_copy(w_hbm.at[ids_v.at[pl.ds(0, S)]], buf, sem).wait()

        bias_regs = tuple(bias_v[pl.ds(u * L, L)] for u in range(UD))

        def accum(j, buf):
            def body(r, accs):
                return tuple(a + buf[r, pl.ds(u * L, L)]
                             for u, a in enumerate(accs))

            accs = lax.fori_loop(0, S, body, bias_regs, unroll=2)
            for u in range(UD):
                acc_v[j, pl.ds(u * L, L)] = accs[u]

        for b in range(_NBUF):
            fire(b, rows[b], sems[b])

        def bag_body(t, carry):
            for b in range(_NBUF):
                j = _NBUF * t + b
                wait(rows[b], sems[b])
                accum(j, rows[b])

                @pl.when(j + _NBUF < BW)
                def _():
                    fire(j + _NBUF, rows[b], sems[b])

            return carry

        lax.fori_loop(0, BW // _NBUF, bag_body, 0)

        pltpu.sync_copy(acc_v, out_hbm.at[pl.ds(base, BW)])

    return k


def kernel(indices, weight, bias):
    B, S = indices.shape
    V, D = weight.shape
    C = V // S
    k = _make_kernel(B, S, D, C)
    return k(indices.astype(jnp.int32), weight, bias)
